# Initial kernel scaffold; baseline (speedup 1.0000x reference)
import functools

import jax
import jax.numpy as jnp
from jax import lax
from jax.experimental import pallas as pl
from jax.experimental.pallas import tpu as pltpu
from jax.experimental.pallas import tpu_sc as plsc

NC = 2
NS = 16
LN = 16
CH = 128

_MESH = plsc.VectorSubcoreMesh(
    core_axis_name="c", subcore_axis_name="s", num_cores=NC, num_subcores=NS)


def _fill_zeros(ref, rows):
  def row(i, carry):
    for k in range(8):
      ref[i, pl.ds(k * 16, 16)] = jnp.zeros((16,), jnp.float32)
    return carry
  lax.fori_loop(0, rows, row, 0)


def _make_deg_kernel(n, kch):
  nz = 1000
  nzt = n // nz
  assert nz * nzt == n and nzt <= NS

  @functools.partial(
      pl.kernel,
      out_type=jax.ShapeDtypeStruct((NC, n), jnp.float32),
      mesh=_MESH,
      scratch_types=[
          pltpu.VMEM((kch, CH), jnp.int32),
          pltpu.VMEM((kch, CH), jnp.float32),
          pltpu.VMEM((1024,), jnp.float32),
          pltpu.VMEM_SHARED((n,), jnp.float32),
      ],
  )
  def deg_kernel(dst_h, ew_h, degp_h, dstv, ewv, zv, acc):
    c = lax.axis_index("c")
    s = lax.axis_index("s")
    t = c * NS + s

    def zrow(i, carry):
      zv[pl.ds(i * 16, 16)] = jnp.zeros((16,), jnp.float32)
      return carry
    lax.fori_loop(0, 64, zrow, 0)

    @pl.when(s < nzt)
    def _():
      pltpu.sync_copy(zv.at[pl.ds(0, nz)], acc.at[pl.ds(s * nz, nz)])
    plsc.subcore_barrier()

    pltpu.sync_copy(dst_h.at[t], dstv)
    pltpu.sync_copy(ew_h.at[t], ewv)

    def chunk(j, carry):
      pltpu.sync_copy(ewv.at[j], acc.at[dstv.at[j]], add=True)
      return carry
    lax.fori_loop(0, kch, chunk, 0)

    plsc.subcore_barrier()
    @pl.when(s < nzt)
    def _():
      pltpu.sync_copy(acc.at[pl.ds(s * nz, nz)], degp_h.at[c, pl.ds(s * nz, nz)])

  return deg_kernel


def _make_agg_kernel(n, d, kch):
  assert d == 128 and n % NS == 0
  npt = n // NS
  nzc = 5
  assert npt % nzc == 0
  zr = npt // nzc
  nvec = d // LN

  @functools.partial(
      pl.kernel,
      out_type=jax.ShapeDtypeStruct((NC, n, d), jnp.float32),
      mesh=_MESH,
      scratch_types=[
          pltpu.VMEM((kch, CH), jnp.int32),
          pltpu.VMEM((kch, CH), jnp.int32),
          pltpu.VMEM((kch, CH), jnp.float32),
          pltpu.VMEM((CH, 128), jnp.float32),
          pltpu.VMEM((125, 128), jnp.float32),
          pltpu.VMEM_SHARED((10000, 128), jnp.float32),
          pltpu.SemaphoreType.DMA,
      ],
  )
  def agg_kernel(g_h, src_h, dst_h, ew_h, a_h,
                 srcv, dstv, ewv, rowbuf, zbuf, acc, sem):
    c = lax.axis_index("c")
    s = lax.axis_index("s")
    t = c * NS + s

    _fill_zeros(zbuf, zr)
    for k in range(nzc):
      pltpu.sync_copy(zbuf, acc.at[pl.ds(s * npt + k * zr, zr)])
    plsc.subcore_barrier()

    pltpu.sync_copy(src_h.at[t], srcv)
    pltpu.sync_copy(dst_h.at[t], dstv)
    pltpu.sync_copy(ew_h.at[t], ewv)

    def chunk(j, carry):
      pltpu.async_copy(g_h.at[srcv.at[j]], rowbuf, sem).wait()
      def edge(e, carry2):
        w = ewv[j, e]
        for k in range(nvec):
          sl = pl.ds(k * 16, 16)
          rowbuf[e, sl] = rowbuf[e, sl] * w
        return carry2
      lax.fori_loop(0, CH, edge, 0)
      pltpu.sync_copy(rowbuf, acc.at[dstv.at[j]], add=True)
      return carry
    lax.fori_loop(0, kch, chunk, 0)

    plsc.subcore_barrier()
    for k in range(nzc):
      r0 = s * npt + k * zr
      pltpu.sync_copy(acc.at[pl.ds(r0, zr)], a_h.at[c, pl.ds(r0, zr)])

  return agg_kernel


def _tc_first(x_ref, w_ref, degp_ref, h_ref, g_ref):
  deg = degp_ref[0, :] + degp_ref[1, :] + 1.0
  dinv = lax.rsqrt(deg)
  h = jnp.dot(x_ref[...], w_ref[...], preferred_element_type=jnp.float32)
  h_ref[...] = h
  g_ref[...] = h * dinv[:, None]


def _tc_mid(a_ref, h_ref, degp_ref, b_ref, w_ref, hout_ref, gout_ref):
  deg = degp_ref[0, :] + degp_ref[1, :] + 1.0
  dinv = lax.rsqrt(deg)
  agg = a_ref[0] + a_ref[1]
  z = (agg * dinv[:, None] + h_ref[...] * (dinv * dinv)[:, None]
       + b_ref[...][None, :])
  xn = jnp.maximum(z, 0.0)
  h2 = jnp.dot(xn, w_ref[...], preferred_element_type=jnp.float32)
  hout_ref[...] = h2
  gout_ref[...] = h2 * dinv[:, None]


def _tc_last(a_ref, h_ref, degp_ref, b_ref, out_ref):
  deg = degp_ref[0, :] + degp_ref[1, :] + 1.0
  dinv = lax.rsqrt(deg)
  agg = a_ref[0] + a_ref[1]
  z = (agg * dinv[:, None] + h_ref[...] * (dinv * dinv)[:, None]
       + b_ref[...][None, :])
  m = jnp.max(z, axis=-1, keepdims=True)
  lse = jnp.log(jnp.sum(jnp.exp(z - m), axis=-1, keepdims=True)) + m
  out_ref[...] = z - lse


def _row_grid_call(body, n, bn, in_specs, out_specs, out_shape):
  return pl.pallas_call(
      body,
      grid=(n // bn,),
      in_specs=in_specs,
      out_specs=out_specs,
      out_shape=out_shape,
  )


@jax.jit
def kernel(x, edge_index, edge_attr, W1, b1, W2, b2, W3, b3):
  n, d = x.shape
  e = edge_index.shape[1]
  h_dim = W1.shape[1]
  c_dim = W3.shape[1]

  nw = NC * NS
  kch = -(-e // (nw * CH))
  ep = nw * kch * CH
  pad = ep - e
  src = jnp.concatenate([edge_index[0], jnp.zeros((pad,), edge_index.dtype)])
  dst = jnp.concatenate([edge_index[1], jnp.zeros((pad,), edge_index.dtype)])
  ew = jnp.concatenate([edge_attr, jnp.zeros((pad,), edge_attr.dtype)])
  src3 = src.reshape(nw, kch, CH)
  dst3 = dst.reshape(nw, kch, CH)
  ew3 = ew.reshape(nw, kch, CH)

  degp = _make_deg_kernel(n, kch)(dst3, ew3)

  agg = _make_agg_kernel(n, h_dim, kch)

  bn = 2000
  f32 = jnp.float32
  w_spec = pl.BlockSpec((d, h_dim), lambda i: (0, 0))
  degp_spec = pl.BlockSpec((NC, bn), lambda i: (0, i))
  row_spec = pl.BlockSpec((bn, h_dim), lambda i: (i, 0))
  a_spec = pl.BlockSpec((NC, bn, h_dim), lambda i: (0, i, 0))
  b_spec = pl.BlockSpec((h_dim,), lambda i: (0,))

  h1, g1 = _row_grid_call(
      _tc_first, n, bn,
      in_specs=[pl.BlockSpec((bn, d), lambda i: (i, 0)), w_spec, degp_spec],
      out_specs=[row_spec, row_spec],
      out_shape=[jax.ShapeDtypeStruct((n, h_dim), f32)] * 2,
  )(x, W1, degp)

  a1 = agg(g1, src3, dst3, ew3)
  h2, g2 = _row_grid_call(
      _tc_mid, n, bn,
      in_specs=[a_spec, row_spec, degp_spec, b_spec, w_spec],
      out_specs=[row_spec, row_spec],
      out_shape=[jax.ShapeDtypeStruct((n, h_dim), f32)] * 2,
  )(a1, h1, degp, b1, W2)

  a2 = agg(g2, src3, dst3, ew3)
  h3, g3 = _row_grid_call(
      _tc_mid, n, bn,
      in_specs=[a_spec, row_spec, degp_spec, b_spec, w_spec],
      out_specs=[row_spec, row_spec],
      out_shape=[jax.ShapeDtypeStruct((n, c_dim), f32)] * 2,
  )(a2, h2, degp, b2, W3)

  a3 = agg(g3, src3, dst3, ew3)
  out = _row_grid_call(
      _tc_last, n, bn,
      in_specs=[a_spec, row_spec, degp_spec, b_spec],
      out_specs=pl.BlockSpec((bn, c_dim), lambda i: (i, 0)),
      out_shape=jax.ShapeDtypeStruct((n, c_dim), f32),
  )(a3, h3, degp, b3)
  return out


# speedup vs baseline: 9.3828x; 9.3828x over previous
import functools

import jax
import jax.numpy as jnp
from jax import lax
from jax.experimental import pallas as pl
from jax.experimental.pallas import tpu as pltpu
from jax.experimental.pallas import tpu_sc as plsc

NC = 2
NS = 16
LN = 16
CH = 128


@functools.cache
def _mesh():
  return plsc.VectorSubcoreMesh(
      core_axis_name="c", subcore_axis_name="s", num_cores=NC, num_subcores=NS)


def _fill_zeros(ref, rows):
  def row(i, carry):
    for k in range(8):
      ref[i, pl.ds(k * 16, 16)] = jnp.zeros((16,), jnp.float32)
    return carry
  lax.fori_loop(0, rows, row, 0)


def _deg_pad(n):
  return -(-n // (NS * 128)) * NS * 128


def _make_deg_kernel(n, kch):
  npad = _deg_pad(n)
  nz = npad // NS

  @functools.partial(
      pl.kernel,
      out_type=jax.ShapeDtypeStruct((NC * npad,), jnp.float32),
      mesh=_mesh(),
      scratch_types=[
          pltpu.VMEM((kch, CH), jnp.int32),
          pltpu.VMEM((kch, CH), jnp.float32),
          pltpu.VMEM((nz,), jnp.float32),
          pltpu.VMEM_SHARED((npad,), jnp.float32),
      ],
  )
  def deg_kernel(dst_h, ew_h, degp_h, dstv, ewv, zv, acc):
    c = lax.axis_index("c")
    s = lax.axis_index("s")
    t = c * NS + s

    def zrow(i, carry):
      zv[pl.ds(i * 16, 16)] = jnp.zeros((16,), jnp.float32)
      return carry
    lax.fori_loop(0, nz // 16, zrow, 0)

    pltpu.sync_copy(zv, acc.at[pl.ds(s * nz, nz)])
    plsc.subcore_barrier()

    pltpu.sync_copy(dst_h.at[t], dstv)
    pltpu.sync_copy(ew_h.at[t], ewv)

    def chunk(j, carry):
      pltpu.sync_copy(ewv.at[j], acc.at[dstv.at[j]], add=True)
      return carry
    lax.fori_loop(0, kch, chunk, 0)

    plsc.subcore_barrier()
    pltpu.sync_copy(acc.at[pl.ds(s * nz, nz)],
                    degp_h.at[pl.ds(c * npad + s * nz, nz)])

  return deg_kernel


def _make_agg_kernel(n, d, kch):
  assert d == 128
  nwt = 10
  npt = n // nwt
  assert npt * nwt == n and npt % 8 == 0
  nvec = d // LN

  @functools.partial(
      pl.kernel,
      out_type=jax.ShapeDtypeStruct((NC, n, d), jnp.float32),
      mesh=_mesh(),
      scratch_types=[
          pltpu.VMEM((kch, CH), jnp.int32),
          pltpu.VMEM((kch, CH), jnp.int32),
          pltpu.VMEM((kch, CH), jnp.float32),
          pltpu.VMEM((CH, d), jnp.float32),
          pltpu.VMEM_SHARED((n, d), jnp.float32),
          pltpu.SemaphoreType.DMA,
      ],
  )
  def agg_kernel(g_h, src_h, dst_h, ew_h, a_h,
                 srcv, dstv, ewv, rowbuf, acc, sem):
    c = lax.axis_index("c")
    s = lax.axis_index("s")
    t = c * NS + s

    _fill_zeros(rowbuf, CH)
    @pl.when(s < nwt)
    def _():
      nfull, rem = divmod(npt, CH)
      for k in range(nfull):
        pltpu.sync_copy(rowbuf, acc.at[pl.ds(s * npt + k * CH, CH)])
      if rem:
        pltpu.sync_copy(rowbuf.at[pl.ds(0, rem)],
                        acc.at[pl.ds(s * npt + nfull * CH, rem)])
    plsc.subcore_barrier()

    pltpu.sync_copy(src_h.at[t], srcv)
    pltpu.sync_copy(dst_h.at[t], dstv)
    pltpu.sync_copy(ew_h.at[t], ewv)

    def chunk(j, carry):
      pltpu.async_copy(g_h.at[srcv.at[j]], rowbuf, sem).wait()
      def grp(g, carry2):
        w16 = ewv[j, pl.ds(g * LN, LN)]
        for el in range(LN):
          e = g * LN + el
          w = w16[el]
          for k in range(nvec):
            sl = pl.ds(k * 16, 16)
            rowbuf[e, sl] = rowbuf[e, sl] * w
        return carry2
      lax.fori_loop(0, CH // LN, grp, 0)
      pltpu.sync_copy(rowbuf, acc.at[dstv.at[j]], add=True)
      return carry
    lax.fori_loop(0, kch, chunk, 0)

    plsc.subcore_barrier()
    @pl.when(s < nwt)
    def _():
      pltpu.sync_copy(acc.at[pl.ds(s * npt, npt)],
                      a_h.at[c, pl.ds(s * npt, npt)])

  return agg_kernel


def _dinv_of(degt_ref):
  deg = degt_ref[:, 0] + degt_ref[:, 1] + 1.0
  return lax.rsqrt(deg)


def _tc_first(x_ref, w_ref, degt_ref, h_ref, g_ref):
  dinv = _dinv_of(degt_ref)
  h = jnp.dot(x_ref[...], w_ref[...], preferred_element_type=jnp.float32)
  h_ref[...] = h
  g_ref[...] = h * dinv[:, None]


def _tc_mid(a_ref, h_ref, degt_ref, b_ref, w_ref, hout_ref, gout_ref):
  dinv = _dinv_of(degt_ref)
  agg = a_ref[0] + a_ref[1]
  z = (agg * dinv[:, None] + h_ref[...] * (dinv * dinv)[:, None]
       + b_ref[...][None, :])
  xn = jnp.maximum(z, 0.0)
  h2 = jnp.dot(xn, w_ref[...], preferred_element_type=jnp.float32)
  hout_ref[...] = h2
  gout_ref[...] = h2 * dinv[:, None]


def _tc_last(a_ref, h_ref, degt_ref, b_ref, out_ref):
  dinv = _dinv_of(degt_ref)
  agg = a_ref[0] + a_ref[1]
  z = (agg * dinv[:, None] + h_ref[...] * (dinv * dinv)[:, None]
       + b_ref[...][None, :])
  m = jnp.max(z, axis=-1, keepdims=True)
  lse = jnp.log(jnp.sum(jnp.exp(z - m), axis=-1, keepdims=True)) + m
  out_ref[...] = z - lse


def _row_grid_call(body, n, bn, in_specs, out_specs, out_shape):
  return pl.pallas_call(
      body,
      grid=(n // bn,),
      in_specs=in_specs,
      out_specs=out_specs,
      out_shape=out_shape,
  )


@jax.jit
def kernel(x, edge_index, edge_attr, W1, b1, W2, b2, W3, b3):
  n, d = x.shape
  e = edge_index.shape[1]
  h_dim = W1.shape[1]
  c_dim = W3.shape[1]

  nw = NC * NS
  kch = -(-e // (nw * CH))
  ep = nw * kch * CH
  pad = ep - e
  src = jnp.concatenate([edge_index[0], jnp.zeros((pad,), edge_index.dtype)])
  dst = jnp.concatenate([edge_index[1], jnp.zeros((pad,), edge_index.dtype)])
  ew = jnp.concatenate([edge_attr, jnp.zeros((pad,), edge_attr.dtype)])
  src3 = src.reshape(nw, kch, CH)
  dst3 = dst.reshape(nw, kch, CH)
  ew3 = ew.reshape(nw, kch, CH)

  npad = _deg_pad(n)
  degp = _make_deg_kernel(n, kch)(dst3, ew3).reshape(NC, npad)[:, :n]
  degt = degp.T

  agg = _make_agg_kernel(n, h_dim, kch)

  bn = 2000
  f32 = jnp.float32
  w_spec = pl.BlockSpec((d, h_dim), lambda i: (0, 0))
  degp_spec = pl.BlockSpec((bn, NC), lambda i: (i, 0))
  row_spec = pl.BlockSpec((bn, h_dim), lambda i: (i, 0))
  a_spec = pl.BlockSpec((NC, bn, h_dim), lambda i: (0, i, 0))
  b_spec = pl.BlockSpec((h_dim,), lambda i: (0,))

  h1, g1 = _row_grid_call(
      _tc_first, n, bn,
      in_specs=[pl.BlockSpec((bn, d), lambda i: (i, 0)), w_spec, degp_spec],
      out_specs=[row_spec, row_spec],
      out_shape=[jax.ShapeDtypeStruct((n, h_dim), f32)] * 2,
  )(x, W1, degt)

  a1 = agg(g1, src3, dst3, ew3)
  h2, g2 = _row_grid_call(
      _tc_mid, n, bn,
      in_specs=[a_spec, row_spec, degp_spec, b_spec, w_spec],
      out_specs=[row_spec, row_spec],
      out_shape=[jax.ShapeDtypeStruct((n, h_dim), f32)] * 2,
  )(a1, h1, degt, b1, W2)

  a2 = agg(g2, src3, dst3, ew3)
  h3, g3 = _row_grid_call(
      _tc_mid, n, bn,
      in_specs=[a_spec, row_spec, degp_spec, b_spec, w_spec],
      out_specs=[row_spec, row_spec],
      out_shape=[jax.ShapeDtypeStruct((n, c_dim), f32)] * 2,
  )(a2, h2, degt, b2, W3)

  a3 = agg(g3, src3, dst3, ew3)
  out = _row_grid_call(
      _tc_last, n, bn,
      in_specs=[a_spec, row_spec, degp_spec, b_spec],
      out_specs=pl.BlockSpec((bn, c_dim), lambda i: (i, 0)),
      out_shape=jax.ShapeDtypeStruct((n, c_dim), f32),
  )(a3, h3, degt, b3)
  return out
